# Initial kernel scaffold; baseline (speedup 1.0000x reference)
import jax
import jax.numpy as jnp
from jax.experimental import pallas as pl


def kernel(x_float, x_binary, edge_index, W_l, W_r, att, bias):
    raise NotImplementedError("write your pallas kernel here")


# speedup vs baseline: 11.5982x; 11.5982x over previous
import functools

import jax
import jax.numpy as jnp
from jax import lax
from jax.experimental import pallas as pl
from jax.experimental.pallas import tpu as pltpu
from jax.experimental.pallas import tpu_sc as plsc

N = 10000
E = 320000
FF = 128
C = 128
NEG = 0.2

NC = 2
NS = 16
NW = NC * NS
EPW = E // NW
CH = 80
NCH = EPW // CH
G = CH // 16


def _mm_body(xf_ref, xb_ref, wlf_ref, wlb_ref, wrf_ref, wrb_ref,
             xl_ref, xr_ref):
    xf = xf_ref[...]
    xb = xb_ref[...]
    xl_ref[...] = (jnp.dot(xf, wlf_ref[...], preferred_element_type=jnp.float32)
                   + xb * wlb_ref[...])
    xr_ref[...] = (jnp.dot(xf, wrf_ref[...], preferred_element_type=jnp.float32)
                   + xb * wrb_ref[...])


def _mm(xf, xb, wlf, wlb, wrf, wrb):
    br = 2000
    grid = N // br
    return pl.pallas_call(
        _mm_body,
        grid=(grid,),
        in_specs=[
            pl.BlockSpec((br, FF), lambda i: (i, 0)),
            pl.BlockSpec((br, 1), lambda i: (i, 0)),
            pl.BlockSpec((FF, C), lambda i: (0, 0)),
            pl.BlockSpec((1, C), lambda i: (0, 0)),
            pl.BlockSpec((FF, C), lambda i: (0, 0)),
            pl.BlockSpec((1, C), lambda i: (0, 0)),
        ],
        out_specs=[
            pl.BlockSpec((br, C), lambda i: (i, 0)),
            pl.BlockSpec((br, C), lambda i: (i, 0)),
        ],
        out_shape=[
            jax.ShapeDtypeStruct((N, C), jnp.float32),
            jax.ShapeDtypeStruct((N, C), jnp.float32),
        ],
    )(xf, xb, wlf, wlb, wrf, wrb)


_MESH = plsc.VectorSubcoreMesh(core_axis_name="c", subcore_axis_name="s")


@functools.partial(
    pl.kernel,
    out_type=(jax.ShapeDtypeStruct((NW, NCH, CH), jnp.float32),
              jax.ShapeDtypeStruct((NC * N,), jnp.float32)),
    mesh=_MESH,
    compiler_params=pltpu.CompilerParams(needs_layout_passes=False),
    scratch_types=[
        pltpu.VMEM((NCH, CH), jnp.int32),
        pltpu.VMEM((NCH, CH), jnp.int32),
        pltpu.VMEM((CH, C), jnp.float32),
        pltpu.VMEM((CH, C), jnp.float32),
        pltpu.VMEM((CH, C), jnp.float32),
        pltpu.VMEM((CH, C), jnp.float32),
        pltpu.VMEM((NCH, CH), jnp.float32),
        pltpu.VMEM((C,), jnp.float32),
        pltpu.VMEM((16, 16), jnp.float32),
        pltpu.VMEM((2000,), jnp.float32),
        pltpu.VMEM_SHARED((N,), jnp.float32),
        pltpu.SemaphoreType.DMA,
        pltpu.SemaphoreType.DMA,
        pltpu.SemaphoreType.DMA,
        pltpu.SemaphoreType.DMA,
    ],
)
def _k2(xl_hbm, xr_hbm, src_hbm, dst_hbm, att_hbm, p_out, den_out,
        src_v, dst_v, la, lb, ra, rb, p_v, att_v, tbuf, zbuf, den_sh,
        sla, slb, sra, srb):
    cid = lax.axis_index("c")
    sid = lax.axis_index("s")
    wid = cid * NS + sid

    pltpu.sync_copy(src_hbm.at[wid], src_v)
    pltpu.sync_copy(dst_hbm.at[wid], dst_v)
    pltpu.sync_copy(att_hbm, att_v)

    @pl.when(sid < 5)
    def _():
        def zfill(i, _):
            zbuf[pl.ds(i * 16, 16)] = jnp.zeros((16,), jnp.float32)
            return 0
        lax.fori_loop(0, 125, zfill, 0)
        pltpu.sync_copy(zbuf, den_sh.at[pl.ds(sid * 2000, 2000)])

    plsc.subcore_barrier()

    atts = [att_v[pl.ds(k * 16, 16)] for k in range(8)]
    lanes = lax.iota(jnp.int32, 16)

    def start(c, bl, br_, sl, sr):
        pltpu.async_copy(xl_hbm.at[src_v.at[c]], bl, sl)
        pltpu.async_copy(xr_hbm.at[dst_v.at[c]], br_, sr)

    def wait(bl, br_, sl, sr):
        pltpu.make_async_copy(xl_hbm.at[src_v.at[0]], bl, sl).wait()
        pltpu.make_async_copy(xr_hbm.at[dst_v.at[0]], br_, sr).wait()

    def compute(c, bl, br_):
        def grp(g, _):
            for j in range(16):
                e = g * 16 + j
                acc = jnp.zeros((16,), jnp.float32)
                for k in range(8):
                    sl = pl.ds(k * 16, 16)
                    t = bl[e, sl] + br_[e, sl]
                    t = jnp.maximum(t, t * NEG)
                    acc = acc + t * atts[k]
                plsc.store_scatter(tbuf, [lanes, jnp.full((16,), j, jnp.int32)], acc)
            alpha = tbuf[0, :]
            for l in range(1, 16):
                alpha = alpha + tbuf[l, :]
            p_v[c, pl.ds(g * 16, 16)] = jnp.exp(alpha)
            return 0
        lax.fori_loop(0, G, grp, 0)
        pltpu.sync_copy(p_v.at[c], den_sh.at[dst_v.at[c]], add=True)

    start(0, la, ra, sla, sra)

    def body(i, _):
        ca = i * 2
        cb = ca + 1
        wait(la, ra, sla, sra)
        start(cb, lb, rb, slb, srb)
        compute(ca, la, ra)
        wait(lb, rb, slb, srb)
        start(cb + 1, la, ra, sla, sra)
        compute(cb, lb, rb)
        return 0

    lax.fori_loop(0, NCH // 2, body, 0)
    wait(la, ra, sla, sra)
    compute(NCH - 1, la, ra)

    pltpu.sync_copy(p_v, p_out.at[wid])

    plsc.subcore_barrier()

    @pl.when(sid < 5)
    def _():
        pltpu.sync_copy(den_sh.at[pl.ds(sid * 2000, 2000)], zbuf)
        pltpu.sync_copy(zbuf, den_out.at[pl.ds(cid * N + sid * 2000, 2000)])


@functools.partial(
    pl.kernel,
    out_type=jax.ShapeDtypeStruct((NC, N, C), jnp.float32),
    mesh=_MESH,
    compiler_params=pltpu.CompilerParams(needs_layout_passes=False),
    scratch_types=[
        pltpu.VMEM((NCH, CH), jnp.int32),
        pltpu.VMEM((CH,), jnp.int32),
        pltpu.VMEM((CH,), jnp.int32),
        pltpu.VMEM((CH,), jnp.int32),
        pltpu.VMEM((CH,), jnp.int32),
        pltpu.VMEM((CH,), jnp.float32),
        pltpu.VMEM((CH,), jnp.float32),
        pltpu.VMEM((CH,), jnp.float32),
        pltpu.VMEM((CH,), jnp.float32),
        pltpu.VMEM((CH,), jnp.float32),
        pltpu.VMEM((CH,), jnp.float32),
        pltpu.VMEM((CH, C), jnp.float32),
        pltpu.VMEM((CH, C), jnp.float32),
        pltpu.VMEM_SHARED((N, C), jnp.float32),
        pltpu.SemaphoreType.DMA,
        pltpu.SemaphoreType.DMA,
        pltpu.SemaphoreType.DMA,
        pltpu.SemaphoreType.DMA,
        pltpu.SemaphoreType.DMA,
        pltpu.SemaphoreType.DMA,
    ],
)
def _k3(xl_hbm, src_hbm, dst_hbm, dstn_hbm, p_hbm, den_hbm, out_hbm,
        src_v, dia, dib, dna, dnb, pa, pb, d0a, d0b, d1a, d1b, ba, bb,
        out_sh, sia, sib, sga, sgb, sda, sdb):
    cid = lax.axis_index("c")
    sid = lax.axis_index("s")
    wid = cid * NS + sid
    ebase = wid * EPW

    pltpu.sync_copy(src_hbm.at[wid], src_v)

    def zfill(i, _):
        for k in range(8):
            ba[i, pl.ds(k * 16, 16)] = jnp.zeros((16,), jnp.float32)
        return 0

    @pl.when(sid < 10)
    def _():
        lax.fori_loop(0, CH, zfill, 0)
        for s in range(12):
            pltpu.sync_copy(ba, out_sh.at[pl.ds(sid * 1000 + s * 80, 80)])
        pltpu.sync_copy(ba.at[pl.ds(0, 40)], out_sh.at[pl.ds(sid * 1000 + 960, 40)])

    plsc.subcore_barrier()

    def start_idx(c, di, dn, sem):
        pltpu.async_copy(dst_hbm.at[pl.ds(ebase + c * CH, CH)], di, sem)
        pltpu.async_copy(dstn_hbm.at[pl.ds(ebase + c * CH, CH)], dn, sem)

    def wait_idx(di, dn, sem):
        pltpu.make_async_copy(dst_hbm.at[pl.ds(0, CH)], di, sem).wait()
        pltpu.make_async_copy(dstn_hbm.at[pl.ds(0, CH)], dn, sem).wait()

    def start_rows(c, p_b, rows, sem):
        pltpu.async_copy(p_hbm.at[pl.ds(ebase + c * CH, CH)], p_b, sem)
        pltpu.async_copy(xl_hbm.at[src_v.at[c]], rows, sem)

    def wait_rows(p_b, rows, sem):
        pltpu.make_async_copy(p_hbm.at[pl.ds(0, CH)], p_b, sem).wait()
        pltpu.make_async_copy(xl_hbm.at[src_v.at[0]], rows, sem).wait()

    def start_den(di, dn, d0, d1, sem):
        pltpu.async_copy(den_hbm.at[di], d0, sem)
        pltpu.async_copy(den_hbm.at[dn], d1, sem)

    def wait_den(di, dn, d0, d1, sem):
        pltpu.make_async_copy(den_hbm.at[di], d0, sem).wait()
        pltpu.make_async_copy(den_hbm.at[dn], d1, sem).wait()

    def scale_scatter(di, p_b, d0, d1, rows):
        def grp(g, _):
            sl = pl.ds(g * 16, 16)
            w16 = p_b[sl] / (d0[sl] + d1[sl] + 1e-16)
            for j in range(16):
                e = g * 16 + j
                we = w16[j]
                for k in range(8):
                    ssl = pl.ds(k * 16, 16)
                    rows[e, ssl] = rows[e, ssl] * we
            return 0
        lax.fori_loop(0, G, grp, 0)
        pltpu.sync_copy(rows, out_sh.at[di], add=True)

    start_idx(0, dia, dna, sia)
    start_idx(1, dib, dnb, sib)
    start_rows(0, pa, ba, sga)
    wait_idx(dia, dna, sia)
    start_den(dia, dna, d0a, d1a, sda)

    def body(i, _):
        ca = i * 2
        cb = ca + 1
        start_rows(cb, pb, bb, sgb)
        wait_idx(dib, dnb, sib)
        start_den(dib, dnb, d0b, d1b, sdb)
        wait_rows(pa, ba, sga)
        wait_den(dia, dna, d0a, d1a, sda)
        scale_scatter(dia, pa, d0a, d1a, ba)
        start_idx(ca + 2, dia, dna, sia)
        start_rows(ca + 2, pa, ba, sga)
        wait_idx(dia, dna, sia)
        start_den(dia, dna, d0a, d1a, sda)
        wait_rows(pb, bb, sgb)
        wait_den(dib, dnb, d0b, d1b, sdb)
        scale_scatter(dib, pb, d0b, d1b, bb)

        @pl.when(cb + 2 < NCH)
        def _():
            start_idx(cb + 2, dib, dnb, sib)
        return 0

    lax.fori_loop(0, NCH // 2, body, 0)
    wait_rows(pa, ba, sga)
    wait_den(dia, dna, d0a, d1a, sda)
    scale_scatter(dia, pa, d0a, d1a, ba)

    plsc.subcore_barrier()

    @pl.when(sid < 10)
    def _():
        for s in range(12):
            pltpu.sync_copy(out_sh.at[pl.ds(sid * 1000 + s * 80, 80)], ba)
            pltpu.sync_copy(ba, out_hbm.at[cid, pl.ds(sid * 1000 + s * 80, 80)])
        pltpu.sync_copy(out_sh.at[pl.ds(sid * 1000 + 960, 40)], ba.at[pl.ds(0, 40)])
        pltpu.sync_copy(ba.at[pl.ds(0, 40)], out_hbm.at[cid, pl.ds(sid * 1000 + 960, 40)])


def _comb_body(a_ref, b_ref, bias_ref, o_ref):
    o_ref[...] = a_ref[...] + b_ref[...] + bias_ref[...]


def _comb(a, b, bias2d):
    br = 2000
    return pl.pallas_call(
        _comb_body,
        grid=(N // br,),
        in_specs=[
            pl.BlockSpec((br, C), lambda i: (i, 0)),
            pl.BlockSpec((br, C), lambda i: (i, 0)),
            pl.BlockSpec((1, C), lambda i: (0, 0)),
        ],
        out_specs=pl.BlockSpec((br, C), lambda i: (i, 0)),
        out_shape=jax.ShapeDtypeStruct((N, C), jnp.float32),
    )(a, b, bias2d)


def kernel(x_float, x_binary, edge_index, W_l, W_r, att, bias):
    xb = x_binary.reshape(N, 1).astype(jnp.float32)
    xl, xr = _mm(x_float, xb, W_l[:FF], W_l[FF:], W_r[:FF], W_r[FF:])
    src3 = edge_index[0].reshape(NW, NCH, CH).astype(jnp.int32)
    dst3 = edge_index[1].reshape(NW, NCH, CH).astype(jnp.int32)
    attv = att.reshape(C)
    p3, denp = _k2(xl, xr, src3, dst3, attv)
    dst1 = dst3.reshape(E)
    outp = _k3(xl, src3, dst1, dst1 + N, p3.reshape(E), denp)
    return _comb(outp[0], outp[1], bias.reshape(1, C))
